# Initial kernel scaffold; baseline (speedup 1.0000x reference)
import jax
import jax.numpy as jnp
from jax.experimental import pallas as pl


def kernel(input_x, input_centroids):
    raise NotImplementedError("write your pallas kernel here")


# speedup vs baseline: 3.0137x; 3.0137x over previous
import jax
import jax.numpy as jnp
from jax.experimental import pallas as pl
from jax.experimental.pallas import tpu as pltpu

N, K, D = 65536, 1024, 32
BN = 1024
NB = N // BN


def _body(x_ref, c_ref, assign_ref, cent_ref, sdist_ref,
          sums_scr, counts_scr, sacc_scr):
    i = pl.program_id(0)

    @pl.when(i == 0)
    def _init():
        sums_scr[...] = jnp.zeros_like(sums_scr)
        counts_scr[...] = jnp.zeros_like(counts_scr)
        sacc_scr[...] = jnp.zeros_like(sacc_scr)

    x = x_ref[...]
    c = c_ref[...]
    cn = jnp.sum(c * c, axis=1, keepdims=True)
    ca = jnp.concatenate([c, cn], axis=1)
    ones_col = jnp.ones((BN, 1), jnp.float32)
    xa = jnp.concatenate([-2.0 * x, ones_col], axis=1)
    dist = jax.lax.dot_general(
        xa, ca, dimension_numbers=(((1,), (1,)), ((), ())),
        preferred_element_type=jnp.float32,
        precision=jax.lax.Precision.HIGHEST)

    minval = jnp.min(dist, axis=1, keepdims=True)
    iota_k = jax.lax.broadcasted_iota(jnp.int32, (BN, K), 1)
    masked = jnp.where(dist == minval, iota_k, K)
    idx = jnp.min(masked, axis=1, keepdims=True)
    assign_ref[...] = idx

    onehot = (iota_k == idx).astype(jnp.float32)
    sums_scr[...] += jax.lax.dot_general(
        onehot, x, dimension_numbers=(((0,), (0,)), ((), ())),
        preferred_element_type=jnp.float32,
        precision=jax.lax.Precision.HIGHEST)
    counts_scr[...] += jax.lax.dot_general(
        onehot, ones_col, dimension_numbers=(((0,), (0,)), ((), ())),
        preferred_element_type=jnp.float32,
        precision=jax.lax.Precision.HIGHEST)
    xn = jnp.sum(x * x, axis=1, keepdims=True)
    sacc_scr[...] = sacc_scr[...] + jnp.sum(minval + xn)

    @pl.when(i == NB - 1)
    def _finish():
        cent_ref[...] = sums_scr[...] / counts_scr[...]
        sdist_ref[...] = sacc_scr[...]


@jax.jit
def kernel(input_x, input_centroids):
    assign2, cent, sdist = pl.pallas_call(
        _body,
        grid=(NB,),
        in_specs=[
            pl.BlockSpec((BN, D), lambda i: (i, 0)),
            pl.BlockSpec((K, D), lambda i: (0, 0)),
        ],
        out_specs=[
            pl.BlockSpec((BN, 1), lambda i: (i, 0)),
            pl.BlockSpec((K, D), lambda i: (0, 0)),
            pl.BlockSpec((1, 1), lambda i: (0, 0)),
        ],
        out_shape=[
            jax.ShapeDtypeStruct((N, 1), jnp.int32),
            jax.ShapeDtypeStruct((K, D), jnp.float32),
            jax.ShapeDtypeStruct((1, 1), jnp.float32),
        ],
        scratch_shapes=[
            pltpu.VMEM((K, D), jnp.float32),
            pltpu.VMEM((K, 1), jnp.float32),
            pltpu.VMEM((1, 1), jnp.float32),
        ],
    )(input_x, input_centroids)
    assignments = assign2.reshape(N)
    return assignments, cent, sdist[0, 0]
